# baseline (speedup 1.0000x reference)
import functools

import jax
import jax.numpy as jnp
from jax import lax
from jax.experimental import pallas as pl
from jax.experimental.pallas import tpu as pltpu
from jax.experimental.pallas import tpu_sc as plsc

S, H, NH, HD, E, FF = 2048, 768, 12, 64, 64, 384
EPS = 1e-05
BS = 512
BQ = 512
BM = 32
NT = 128
PAD = NT * BM


def _rms(x, w):
    v = jnp.mean(x * x, axis=-1, keepdims=True)
    return (x * lax.rsqrt(v + EPS)) * w


def _rotate_half_cols(q):
    parts = []
    for h in range(NH):
        b = h * HD
        parts.append(-q[:, b + HD // 2:b + HD])
        parts.append(q[:, b:b + HD // 2])
    return jnp.concatenate(parts, axis=1)


def _qkv_body(x_ref, cos_ref, sin_ref, lnw_ref, wq_ref, wk_ref, wv_ref,
              q_ref, k_ref, v_ref):
    x = x_ref[...]
    xn = _rms(x, lnw_ref[...]).astype(jnp.bfloat16)
    cos = cos_ref[...]
    sin = sin_ref[...]
    q = jnp.dot(xn, wq_ref[...], preferred_element_type=jnp.float32)
    k = jnp.dot(xn, wk_ref[...], preferred_element_type=jnp.float32)
    v = jnp.dot(xn, wv_ref[...], preferred_element_type=jnp.float32)
    q_ref[...] = (q * cos + _rotate_half_cols(q) * sin).astype(jnp.bfloat16)
    k_ref[...] = (k * cos + _rotate_half_cols(k) * sin).astype(jnp.bfloat16)
    v_ref[...] = v.astype(jnp.bfloat16)


def _attn_body(q_ref, k_ref, v_ref, o_ref):
    q = q_ref[0]
    k = k_ref[0]
    v = v_ref[0]
    s = lax.dot_general(q, k, (((1,), (1,)), ((), ())),
                        preferred_element_type=jnp.float32)
    m = jnp.max(s, axis=-1, keepdims=True)
    p = jnp.exp(s - m)
    l = jnp.sum(p, axis=-1, keepdims=True)
    o = jnp.dot(p.astype(jnp.bfloat16), v, preferred_element_type=jnp.float32)
    o_ref[0] = (o / l).astype(jnp.bfloat16)


def _post_body(ao_ref, res_ref, plnw_ref, wo_ref, wr_ref,
               h2_ref, xn_ref, ti_ref):
    h2 = res_ref[...] + jnp.dot(ao_ref[...], wo_ref[...],
                                preferred_element_type=jnp.float32)
    h2_ref[...] = h2
    xn = _rms(h2, plnw_ref[...])
    xn_ref[...] = xn
    logits = jnp.dot(xn, wr_ref[...], preferred_element_type=jnp.float32)
    ti_ref[...] = jnp.argmax(logits, axis=-1).astype(jnp.int32)[:, None]


def _ffn_body(eid_ref, xs_ref, wg_ref, wu_ref, wd_ref, ys_ref):
    del eid_ref
    x = xs_ref[...]
    g = jnp.dot(x, wg_ref[0], preferred_element_type=jnp.float32)
    u = jnp.dot(x, wu_ref[0], preferred_element_type=jnp.float32)
    a = (g * (1.0 / (1.0 + jnp.exp(-g)))) * u
    ys_ref[...] = jnp.dot(
        a, wd_ref[0], preferred_element_type=jnp.float32).astype(jnp.bfloat16)


def _final_body(h2_ref, ys_ref, sr_ref, o_ref):
    sr = sr_ref[...]
    iota = lax.broadcasted_iota(jnp.int32, (BS, PAD), 1)
    oh = (iota == sr).astype(jnp.bfloat16)
    moe = jnp.dot(oh, ys_ref[...], preferred_element_type=jnp.float32)
    o_ref[...] = h2_ref[...] + moe


def _gather_rows(table, idx, n_rows):
    d = table.shape[1]
    info = plsc.get_sparse_core_info()
    nc, ns = info.num_cores, info.num_subcores
    nw = nc * ns
    b = n_rows // nw
    mesh = plsc.VectorSubcoreMesh(core_axis_name="c", subcore_axis_name="s")

    @functools.partial(
        pl.kernel, mesh=mesh,
        out_type=jax.ShapeDtypeStruct((n_rows, d), table.dtype),
        scratch_types=[
            pltpu.VMEM((b,), jnp.int32),
            pltpu.VMEM((b, d), table.dtype),
            pltpu.SemaphoreType.DMA,
        ],
    )
    def gk(t_hbm, i_hbm, o_hbm, idx_v, rows_v, sem):
        wid = lax.axis_index("s") * nc + lax.axis_index("c")
        base = wid * b
        pltpu.sync_copy(i_hbm.at[pl.ds(base, b)], idx_v)
        pltpu.async_copy(t_hbm.at[idx_v], rows_v, sem).wait()
        pltpu.sync_copy(rows_v, o_hbm.at[pl.ds(base, b)])

    return gk(table, idx)


def kernel(hidden_states, cos, sin, input_ln_w, Wq, Wk, Wv, Wo, post_ln_w,
           Wr, Wg, Wu, Wd):
    f32 = jnp.float32
    bf16 = jnp.bfloat16
    x0 = hidden_states.reshape(S, H)
    cos_t = jnp.tile(cos.reshape(S, HD), (1, NH))
    sin_t = jnp.tile(sin.reshape(S, HD), (1, NH))
    scale = 1.0 / (HD ** 0.5)

    row_spec = pl.BlockSpec((BS, H), lambda i: (i, 0))
    full_spec = pl.BlockSpec((H, H), lambda i: (0, 0))
    vec_spec = pl.BlockSpec((1, H), lambda i: (0, 0))
    q, k, v = pl.pallas_call(
        _qkv_body,
        grid=(S // BS,),
        in_specs=[row_spec, row_spec, row_spec, vec_spec,
                  full_spec, full_spec, full_spec],
        out_specs=[row_spec, row_spec, row_spec],
        out_shape=[jax.ShapeDtypeStruct((S, H), bf16)] * 3,
    )(x0, cos_t, sin_t, input_ln_w.reshape(1, H),
      (Wq * scale).astype(bf16), Wk.astype(bf16), Wv.astype(bf16))

    q3 = jnp.transpose(q.reshape(S, NH, HD), (1, 0, 2))
    k3 = jnp.transpose(k.reshape(S, NH, HD), (1, 0, 2))
    v3 = jnp.transpose(v.reshape(S, NH, HD), (1, 0, 2))
    qo_spec = pl.BlockSpec((1, BQ, HD), lambda h, i: (h, i, 0))
    kv_spec = pl.BlockSpec((1, S, HD), lambda h, i: (h, 0, 0))
    ao3 = pl.pallas_call(
        _attn_body,
        grid=(NH, S // BQ),
        in_specs=[qo_spec, kv_spec, kv_spec],
        out_specs=qo_spec,
        out_shape=jax.ShapeDtypeStruct((NH, S, HD), bf16),
    )(q3, k3, v3)
    ao = jnp.transpose(ao3, (1, 0, 2)).reshape(S, H)

    h2, xn, ti2 = pl.pallas_call(
        _post_body,
        grid=(S // BS,),
        in_specs=[row_spec, row_spec, vec_spec, full_spec,
                  pl.BlockSpec((H, E), lambda i: (0, 0))],
        out_specs=[row_spec, row_spec, pl.BlockSpec((BS, 1), lambda i: (i, 0))],
        out_shape=[jax.ShapeDtypeStruct((S, H), f32),
                   jax.ShapeDtypeStruct((S, H), f32),
                   jax.ShapeDtypeStruct((S, 1), jnp.int32)],
    )(ao, x0, post_ln_w.reshape(1, H), Wo.astype(bf16), Wr)
    ti = ti2[:, 0]

    oh_i = (ti[:, None] == jnp.arange(E, dtype=jnp.int32)[None, :])
    oh_i = oh_i.astype(jnp.int32)
    counts = jnp.sum(oh_i, axis=0).astype(jnp.int32)
    off = jnp.concatenate([jnp.zeros((1,), jnp.int32),
                           jnp.cumsum(counts)[:-1].astype(jnp.int32)])
    rank = jnp.take_along_axis(
        jnp.cumsum(oh_i, axis=0), ti[:, None], axis=1)[:, 0] - 1
    p_tok = off[ti] + rank.astype(jnp.int32)
    order = jnp.zeros((S,), jnp.int32).at[p_tok].set(
        jnp.arange(S, dtype=jnp.int32))
    nt_e = (counts + BM - 1) // BM
    cum_nt = jnp.cumsum(nt_e).astype(jnp.int32)
    total = cum_nt[-1]
    t_idx = jnp.arange(NT, dtype=jnp.int32)
    e_t = jnp.minimum(
        jnp.searchsorted(cum_nt, t_idx, side="right"), E - 1).astype(jnp.int32)
    j_t = t_idx - (cum_nt[e_t] - nt_e[e_t])
    start_p = off[e_t] + j_t * BM
    valid_len = jnp.clip(counts[e_t] - j_t * BM, 0, BM)
    m = jnp.arange(BM, dtype=jnp.int32)
    p_mat = start_p[:, None] + m[None, :]
    valid = m[None, :] < valid_len[:, None]
    src_token = jnp.where(valid, order[jnp.clip(p_mat, 0, S - 1)], 0)
    src_token = src_token.reshape(PAD).astype(jnp.int32)
    r_idx = jnp.arange(PAD, dtype=jnp.int32)
    tok_or_oob = jnp.where(valid.reshape(PAD), src_token, S)
    src_row = jnp.zeros((S,), jnp.int32).at[tok_or_oob].set(r_idx, mode="drop")
    last_e = e_t[jnp.clip(total - 1, 0, NT - 1)]
    tile_eid = jnp.where(t_idx < total, e_t, last_e).astype(jnp.int32)

    xs = _gather_rows(xn, src_token, PAD)

    grid_spec = pltpu.PrefetchScalarGridSpec(
        num_scalar_prefetch=1,
        grid=(NT,),
        in_specs=[
            pl.BlockSpec((BM, H), lambda i, eid: (i, 0)),
            pl.BlockSpec((1, H, FF), lambda i, eid: (eid[i], 0, 0)),
            pl.BlockSpec((1, H, FF), lambda i, eid: (eid[i], 0, 0)),
            pl.BlockSpec((1, FF, H), lambda i, eid: (eid[i], 0, 0)),
        ],
        out_specs=pl.BlockSpec((BM, H), lambda i, eid: (i, 0)),
    )
    ys = pl.pallas_call(
        _ffn_body,
        grid_spec=grid_spec,
        out_shape=jax.ShapeDtypeStruct((PAD, H), bf16),
    )(tile_eid, xs, Wg, Wu, Wd)

    out = pl.pallas_call(
        _final_body,
        grid=(S // BS,),
        in_specs=[row_spec,
                  pl.BlockSpec((PAD, H), lambda i: (0, 0)),
                  pl.BlockSpec((BS, 1), lambda i: (i, 0))],
        out_specs=row_spec,
        out_shape=jax.ShapeDtypeStruct((S, H), f32),
    )(h2, ys, src_row[:, None])
    return out.reshape(1, S, H)


# speedup vs baseline: 1.2324x; 1.2324x over previous
import functools

import jax
import jax.numpy as jnp
from jax import lax
from jax.experimental import pallas as pl
from jax.experimental.pallas import tpu as pltpu
from jax.experimental.pallas import tpu_sc as plsc

S, H, NH, HD, E, FF = 2048, 768, 12, 64, 64, 384
EPS = 1e-05
BS = 512
BQ = 512
BM = 32
NT = 128
PAD = NT * BM


def _rms(x, w):
    v = jnp.mean(x * x, axis=-1, keepdims=True)
    return (x * lax.rsqrt(v + EPS)) * w


def _rotate_half_cols(q):
    parts = []
    for h in range(NH):
        b = h * HD
        parts.append(-q[:, b + HD // 2:b + HD])
        parts.append(q[:, b:b + HD // 2])
    return jnp.concatenate(parts, axis=1)


def _qkv_body(x_ref, cos_ref, sin_ref, lnw_ref, wq_ref, wk_ref, wv_ref,
              q_ref, k_ref, v_ref):
    x = x_ref[...]
    xn = _rms(x, lnw_ref[...]).astype(jnp.bfloat16)
    cos = jnp.concatenate([cos_ref[...]] * NH, axis=1)
    sin = jnp.concatenate([sin_ref[...]] * NH, axis=1)
    q = jnp.dot(xn, wq_ref[...], preferred_element_type=jnp.float32)
    k = jnp.dot(xn, wk_ref[...], preferred_element_type=jnp.float32)
    v = jnp.dot(xn, wv_ref[...], preferred_element_type=jnp.float32)
    qr = (q * cos + _rotate_half_cols(q) * sin).astype(jnp.bfloat16)
    kr = (k * cos + _rotate_half_cols(k) * sin).astype(jnp.bfloat16)
    vb = v.astype(jnp.bfloat16)
    for h in range(NH):
        q_ref[h] = qr[:, h * HD:(h + 1) * HD]
        k_ref[h] = kr[:, h * HD:(h + 1) * HD]
        v_ref[h] = vb[:, h * HD:(h + 1) * HD]


def _attn_body(q_ref, k_ref, v_ref, o_ref):
    q = q_ref[0]
    k = k_ref[0]
    v = v_ref[0]
    s = lax.dot_general(q, k, (((1,), (1,)), ((), ())),
                        preferred_element_type=jnp.float32)
    m = jnp.max(s, axis=-1, keepdims=True)
    p = jnp.exp(s - m)
    l = jnp.sum(p, axis=-1, keepdims=True)
    o = jnp.dot(p.astype(jnp.bfloat16), v, preferred_element_type=jnp.float32)
    o_ref[0] = (o / l).astype(jnp.bfloat16)


def _post_body(ao_ref, res_ref, plnw_ref, wo_ref, wr_ref,
               h2_ref, xn_ref, ti_ref):
    ao = jnp.concatenate([ao_ref[h] for h in range(NH)], axis=1)
    h2 = res_ref[...] + jnp.dot(ao, wo_ref[...],
                                preferred_element_type=jnp.float32)
    h2_ref[...] = h2
    xn = _rms(h2, plnw_ref[...])
    xn_ref[...] = xn
    logits = jnp.dot(xn, wr_ref[...], preferred_element_type=jnp.float32)
    ti_ref[...] = jnp.argmax(logits, axis=-1).astype(jnp.int32)[:, None]


def _ffn_body(eid_ref, xs_ref, wg_ref, wu_ref, wd_ref, ys_ref):
    del eid_ref
    x = xs_ref[...]
    g = jnp.dot(x, wg_ref[0], preferred_element_type=jnp.float32)
    u = jnp.dot(x, wu_ref[0], preferred_element_type=jnp.float32)
    a = (g * (1.0 / (1.0 + jnp.exp(-g)))) * u
    ys_ref[...] = jnp.dot(
        a, wd_ref[0], preferred_element_type=jnp.float32).astype(jnp.bfloat16)


def _final_body(h2_ref, ys_ref, sr_ref, o_ref):
    sr = sr_ref[...]
    iota = lax.broadcasted_iota(jnp.int32, (BS, PAD), 1)
    oh = (iota == sr).astype(jnp.bfloat16)
    moe = jnp.dot(oh, ys_ref[...], preferred_element_type=jnp.float32)
    o_ref[...] = h2_ref[...] + moe


def _gather_rows(table, idx, n_rows):
    d = table.shape[1]
    info = plsc.get_sparse_core_info()
    nc, ns = info.num_cores, info.num_subcores
    nw = nc * ns
    b = n_rows // nw
    mesh = plsc.VectorSubcoreMesh(core_axis_name="c", subcore_axis_name="s")

    @functools.partial(
        pl.kernel, mesh=mesh,
        out_type=jax.ShapeDtypeStruct((n_rows, d), table.dtype),
        scratch_types=[
            pltpu.VMEM((b,), jnp.int32),
            pltpu.VMEM((b, d), table.dtype),
            pltpu.SemaphoreType.DMA,
        ],
    )
    def gk(t_hbm, i_hbm, o_hbm, idx_v, rows_v, sem):
        wid = lax.axis_index("s") * nc + lax.axis_index("c")
        base = wid * b
        pltpu.sync_copy(i_hbm.at[pl.ds(base, b)], idx_v)
        pltpu.async_copy(t_hbm.at[idx_v], rows_v, sem).wait()
        pltpu.sync_copy(rows_v, o_hbm.at[pl.ds(base, b)])

    return gk(table, idx)


def kernel(hidden_states, cos, sin, input_ln_w, Wq, Wk, Wv, Wo, post_ln_w,
           Wr, Wg, Wu, Wd):
    f32 = jnp.float32
    bf16 = jnp.bfloat16
    x0 = hidden_states.reshape(S, H)
    scale = 1.0 / (HD ** 0.5)

    row_spec = pl.BlockSpec((BS, H), lambda i: (i, 0))
    full_spec = pl.BlockSpec((H, H), lambda i: (0, 0))
    vec_spec = pl.BlockSpec((1, H), lambda i: (0, 0))
    rope_spec = pl.BlockSpec((BS, HD), lambda i: (i, 0))
    head_spec = pl.BlockSpec((NH, BS, HD), lambda i: (0, i, 0))
    q3, k3, v3 = pl.pallas_call(
        _qkv_body,
        grid=(S // BS,),
        in_specs=[row_spec, rope_spec, rope_spec, vec_spec,
                  full_spec, full_spec, full_spec],
        out_specs=[head_spec, head_spec, head_spec],
        out_shape=[jax.ShapeDtypeStruct((NH, S, HD), bf16)] * 3,
    )(x0, cos.reshape(S, HD), sin.reshape(S, HD), input_ln_w.reshape(1, H),
      (Wq * scale).astype(bf16), Wk.astype(bf16), Wv.astype(bf16))

    qo_spec = pl.BlockSpec((1, BQ, HD), lambda h, i: (h, i, 0))
    kv_spec = pl.BlockSpec((1, S, HD), lambda h, i: (h, 0, 0))
    ao3 = pl.pallas_call(
        _attn_body,
        grid=(NH, S // BQ),
        in_specs=[qo_spec, kv_spec, kv_spec],
        out_specs=qo_spec,
        out_shape=jax.ShapeDtypeStruct((NH, S, HD), bf16),
    )(q3, k3, v3)

    h2, xn, ti2 = pl.pallas_call(
        _post_body,
        grid=(S // BS,),
        in_specs=[head_spec, row_spec, vec_spec, full_spec,
                  pl.BlockSpec((H, E), lambda i: (0, 0))],
        out_specs=[row_spec, row_spec, pl.BlockSpec((BS, 1), lambda i: (i, 0))],
        out_shape=[jax.ShapeDtypeStruct((S, H), f32),
                   jax.ShapeDtypeStruct((S, H), f32),
                   jax.ShapeDtypeStruct((S, 1), jnp.int32)],
    )(ao3, x0, post_ln_w.reshape(1, H), Wo.astype(bf16), Wr)
    ti = ti2[:, 0]

    order = jnp.argsort(ti).astype(jnp.int32)
    counts = jnp.bincount(ti, length=E).astype(jnp.int32)
    off = jnp.concatenate([jnp.zeros((1,), jnp.int32),
                           jnp.cumsum(counts)[:-1].astype(jnp.int32)])
    nt_e = (counts + BM - 1) // BM
    cum_nt = jnp.cumsum(nt_e).astype(jnp.int32)
    total = cum_nt[-1]
    t_idx = jnp.arange(NT, dtype=jnp.int32)
    e_t = jnp.minimum(
        jnp.searchsorted(cum_nt, t_idx, side="right"), E - 1).astype(jnp.int32)
    j_t = t_idx - (cum_nt[e_t] - nt_e[e_t])
    start_p = off[e_t] + j_t * BM
    valid_len = jnp.clip(counts[e_t] - j_t * BM, 0, BM)
    m = jnp.arange(BM, dtype=jnp.int32)
    p_mat = start_p[:, None] + m[None, :]
    valid = m[None, :] < valid_len[:, None]
    src_token = jnp.where(valid, order[jnp.clip(p_mat, 0, S - 1)], 0)
    src_token = src_token.reshape(PAD).astype(jnp.int32)
    r_idx = jnp.arange(PAD, dtype=jnp.int32)
    tok_or_oob = jnp.where(valid.reshape(PAD), src_token, S)
    src_row = jnp.zeros((S,), jnp.int32).at[tok_or_oob].set(r_idx, mode="drop")
    last_e = e_t[jnp.clip(total - 1, 0, NT - 1)]
    tile_eid = jnp.where(t_idx < total, e_t, last_e).astype(jnp.int32)

    xs = _gather_rows(xn, src_token, PAD)

    grid_spec = pltpu.PrefetchScalarGridSpec(
        num_scalar_prefetch=1,
        grid=(NT,),
        in_specs=[
            pl.BlockSpec((BM, H), lambda i, eid: (i, 0)),
            pl.BlockSpec((1, H, FF), lambda i, eid: (eid[i], 0, 0)),
            pl.BlockSpec((1, H, FF), lambda i, eid: (eid[i], 0, 0)),
            pl.BlockSpec((1, FF, H), lambda i, eid: (eid[i], 0, 0)),
        ],
        out_specs=pl.BlockSpec((BM, H), lambda i, eid: (i, 0)),
    )
    ys = pl.pallas_call(
        _ffn_body,
        grid_spec=grid_spec,
        out_shape=jax.ShapeDtypeStruct((PAD, H), bf16),
    )(tile_eid, xs, Wg, Wu, Wd)

    out = pl.pallas_call(
        _final_body,
        grid=(S // BS,),
        in_specs=[row_spec,
                  pl.BlockSpec((PAD, H), lambda i: (0, 0)),
                  pl.BlockSpec((BS, 1), lambda i: (i, 0))],
        out_specs=row_spec,
        out_shape=jax.ShapeDtypeStruct((S, H), f32),
    )(h2, ys, src_row[:, None])
    return out.reshape(1, S, H)
